# Initial kernel scaffold; baseline (speedup 1.0000x reference)
import jax
import jax.numpy as jnp
from jax.experimental import pallas as pl


def kernel(x, edge_index, samples, Wz, bz, Wr, br, Wh, bh, Wlz, blz, Wlr, blr, Wlh, blh, Wc1, bc1, Wc2, bc2):
    raise NotImplementedError("write your pallas kernel here")


# speedup vs baseline: 64.7931x; 64.7931x over previous
import functools

import jax
import jax.numpy as jnp
from jax import lax
from jax.experimental import pallas as pl
from jax.experimental.pallas import tpu as pltpu
from jax.experimental.pallas import tpu_sc as plsc

N = 100000
E = 3200000
S = 100000
OUT_CH = 32
HID = 128

NC = 2
NS = 16
NW = NC * NS
CH = 128
KB = 8

NP = 100352
RT = NP // NS
EP = NW * 98 * KB * CH
ROWS_E = EP // CH
RPT_E = ROWS_E // NW
BPT_E = RPT_E // KB
KS = 5
SP = NW * 25 * CH
ROWS_S = SP // CH
RPT_S = ROWS_S // NW

_sc_mesh = plsc.VectorSubcoreMesh(core_axis_name="c", subcore_axis_name="s")
_sc_params = pltpu.CompilerParams(use_tc_tiling_on_sc=False)


@functools.partial(
    pl.kernel,
    out_type=jax.ShapeDtypeStruct((NC, NP), jnp.float32),
    mesh=_sc_mesh,
    compiler_params=_sc_params,
    scratch_types=[
        pltpu.VMEM((KB, CH), jnp.int32),
        pltpu.VMEM((CH,), jnp.float32),
        pltpu.VMEM_SHARED((NP,), jnp.float32),
        pltpu.SemaphoreType.DMA,
    ],
)
def _deg_sc(dst_hbm, zeros_hbm, out_hbm, idx_v, ones_v, deg_sh, sem):
    cid = lax.axis_index("c")
    sid = lax.axis_index("s")
    wid = sid * NC + cid
    for i in range(CH // 16):
        ones_v[pl.ds(i * 16, 16)] = jnp.full((16,), 1.0, jnp.float32)
    pltpu.sync_copy(zeros_hbm.at[pl.ds(sid * RT, RT)],
                    deg_sh.at[pl.ds(sid * RT, RT)])
    plsc.subcore_barrier()
    base = wid * RPT_E

    def body(g, carry):
        row = base + g * KB
        pltpu.sync_copy(dst_hbm.at[pl.ds(row, KB)], idx_v)
        descs = [
            pltpu.async_copy(ones_v, deg_sh.at[idx_v.at[j]], sem, add=True)
            for j in range(KB)
        ]
        for d in descs:
            d.wait()
        return carry

    lax.fori_loop(0, BPT_E, body, 0)
    plsc.subcore_barrier()
    pltpu.sync_copy(deg_sh.at[pl.ds(sid * RT, RT)],
                    out_hbm.at[cid, pl.ds(sid * RT, RT)])


@functools.partial(
    pl.kernel,
    out_type=[jax.ShapeDtypeStruct((NC, NP), jnp.float32),
              jax.ShapeDtypeStruct((NC, NP), jnp.float32)],
    mesh=_sc_mesh,
    compiler_params=_sc_params,
    scratch_types=[
        pltpu.VMEM((KB, CH), jnp.int32),
        pltpu.VMEM((KB, CH), jnp.int32),
        pltpu.VMEM((KB, CH), jnp.float32),
        pltpu.VMEM((KB, CH), jnp.float32),
        pltpu.VMEM_SHARED((NP,), jnp.float32),
        pltpu.VMEM_SHARED((NP,), jnp.float32),
        pltpu.SemaphoreType.DMA,
        pltpu.SemaphoreType.DMA,
    ],
)
def _edge_sc(src_hbm, dst_hbm, y0_hbm, y1_hbm, zeros_hbm, t0_out, t1_out,
             sidx_v, didx_v, v0_v, v1_v, t0_sh, t1_sh, gsem, ssem):
    cid = lax.axis_index("c")
    sid = lax.axis_index("s")
    wid = sid * NC + cid
    pltpu.sync_copy(zeros_hbm.at[pl.ds(sid * RT, RT)],
                    t0_sh.at[pl.ds(sid * RT, RT)])
    pltpu.sync_copy(zeros_hbm.at[pl.ds(sid * RT, RT)],
                    t1_sh.at[pl.ds(sid * RT, RT)])
    plsc.subcore_barrier()
    base = wid * RPT_E

    def body(g, carry):
        row = base + g * KB
        pltpu.sync_copy(src_hbm.at[pl.ds(row, KB)], sidx_v)
        pltpu.sync_copy(dst_hbm.at[pl.ds(row, KB)], didx_v)
        gd = []
        for j in range(KB):
            gd.append(pltpu.async_copy(y0_hbm.at[sidx_v.at[j]],
                                       v0_v.at[j], gsem))
            gd.append(pltpu.async_copy(y1_hbm.at[sidx_v.at[j]],
                                       v1_v.at[j], gsem))
        for d in gd:
            d.wait()
        sd = []
        for j in range(KB):
            sd.append(pltpu.async_copy(v0_v.at[j], t0_sh.at[didx_v.at[j]],
                                       ssem, add=True))
            sd.append(pltpu.async_copy(v1_v.at[j], t1_sh.at[didx_v.at[j]],
                                       ssem, add=True))
        for d in sd:
            d.wait()
        return carry

    lax.fori_loop(0, BPT_E, body, 0)
    plsc.subcore_barrier()
    pltpu.sync_copy(t0_sh.at[pl.ds(sid * RT, RT)],
                    t0_out.at[cid, pl.ds(sid * RT, RT)])
    pltpu.sync_copy(t1_sh.at[pl.ds(sid * RT, RT)],
                    t1_out.at[cid, pl.ds(sid * RT, RT)])


@functools.partial(
    pl.kernel,
    out_type=jax.ShapeDtypeStruct((2, SP, OUT_CH), jnp.float32),
    mesh=_sc_mesh,
    compiler_params=_sc_params,
    scratch_types=[
        pltpu.VMEM((KS, CH), jnp.int32),
        pltpu.VMEM((KS * CH, OUT_CH), jnp.float32),
        pltpu.SemaphoreType.DMA,
    ],
)
def _sample_sc(sidx_hbm, out_tab_hbm, o_hbm, idx_v, r_v, sem):
    cid = lax.axis_index("c")
    sid = lax.axis_index("s")
    wid = sid * NC + cid
    base = wid * RPT_S
    for col in range(2):
        def body(b, carry):
            row = base + b * KS
            pltpu.sync_copy(sidx_hbm.at[col, pl.ds(row, KS)], idx_v)
            gd = [
                pltpu.async_copy(out_tab_hbm.at[idx_v.at[j]],
                                 r_v.at[pl.ds(j * CH, CH)], sem)
                for j in range(KS)
            ]
            for d in gd:
                d.wait()
            pltpu.sync_copy(r_v, o_hbm.at[col, pl.ds(row * CH, KS * CH)])
            return carry

        lax.fori_loop(0, RPT_S // KS, body, 0)


_BLK = 2048


def _prep_body(p0_ref, p1_ref, x_ref, y0_ref, y1_ref, dinv_ref):
    deg = 1.0 + p0_ref[...] + p1_ref[...]
    dinv = lax.rsqrt(deg)
    y0_ref[...] = x_ref[:, 0:1] * dinv
    y1_ref[...] = x_ref[:, 1:2] * dinv
    dinv_ref[...] = dinv


def _combine_body(t00_ref, t01_ref, t10_ref, t11_ref, y0_ref, y1_ref,
                  dinv_ref, Wz_ref, bz_ref, Wlz_ref, blz_ref,
                  Wh_ref, bh_ref, Wlh_ref, blh_ref, out_ref):
    dinv = dinv_ref[...]
    a0 = (t00_ref[...] + t01_ref[...] + y0_ref[...]) * dinv
    a1 = (t10_ref[...] + t11_ref[...] + y1_ref[...]) * dinv
    cz = a0 * Wz_ref[0:1, :] + a1 * Wz_ref[1:2, :] + bz_ref[...]
    ch = a0 * Wh_ref[0:1, :] + a1 * Wh_ref[1:2, :] + bh_ref[...]
    zin = jnp.dot(cz, Wlz_ref[...], preferred_element_type=jnp.float32) \
        + blz_ref[...]
    hin = jnp.dot(ch, Wlh_ref[...], preferred_element_type=jnp.float32) \
        + blh_ref[...]
    out_ref[...] = (1.0 - jax.nn.sigmoid(zin)) * jnp.tanh(hin)


def _head_body(e0_ref, e1_ref, Wc1_ref, bc1_ref, Wc2_ref, bc2_ref,
               pred_ref):
    ie = e0_ref[...] * e1_ref[...]
    h1 = jax.nn.relu(
        jnp.dot(ie, Wc1_ref[...], preferred_element_type=jnp.float32)
        + bc1_ref[...])
    pred_ref[...] = (
        jnp.dot(h1, Wc2_ref[...], preferred_element_type=jnp.float32)
        + bc2_ref[...])


def _full(shape):
    return pl.BlockSpec(shape, lambda i: (0,) * len(shape))


def _rows(shape):
    return pl.BlockSpec(shape, lambda i: (i,) + (0,) * (len(shape) - 1))


def kernel(x, edge_index, samples, Wz, bz, Wr, br, Wh, bh,
           Wlz, blz, Wlr, blr, Wlh, blh, Wc1, bc1, Wc2, bc2):
    f32 = jnp.float32

    pad_idx = N + (jnp.arange(EP - E, dtype=jnp.int32) % 256)
    src = jnp.concatenate([edge_index[0], pad_idx]).reshape(ROWS_E, CH)
    dst = jnp.concatenate([edge_index[1], pad_idx]).reshape(ROWS_E, CH)
    xp = jnp.concatenate([x, jnp.zeros((NP - N, 2), f32)])
    zeros1 = jnp.zeros((NP,), f32)

    degp = _deg_sc(dst, zeros1)

    y0, y1, dinv = pl.pallas_call(
        _prep_body,
        grid=(NP // _BLK,),
        in_specs=[_rows((_BLK, 1)), _rows((_BLK, 1)), _rows((_BLK, 2))],
        out_specs=[_rows((_BLK, 1)), _rows((_BLK, 1)), _rows((_BLK, 1))],
        out_shape=[jax.ShapeDtypeStruct((NP, 1), f32),
                   jax.ShapeDtypeStruct((NP, 1), f32),
                   jax.ShapeDtypeStruct((NP, 1), f32)],
    )(degp[0].reshape(NP, 1), degp[1].reshape(NP, 1), xp)

    t0p, t1p = _edge_sc(src, dst, y0.reshape(NP), y1.reshape(NP), zeros1)

    out_full = pl.pallas_call(
        _combine_body,
        grid=(NP // _BLK,),
        in_specs=[_rows((_BLK, 1))] * 7
        + [_full((2, OUT_CH)), _full((1, OUT_CH)),
           _full((OUT_CH, OUT_CH)), _full((1, OUT_CH)),
           _full((2, OUT_CH)), _full((1, OUT_CH)),
           _full((OUT_CH, OUT_CH)), _full((1, OUT_CH))],
        out_specs=[_rows((_BLK, OUT_CH))],
        out_shape=[jax.ShapeDtypeStruct((NP, OUT_CH), f32)],
    )(t0p[0].reshape(NP, 1), t0p[1].reshape(NP, 1),
      t1p[0].reshape(NP, 1), t1p[1].reshape(NP, 1),
      y0, y1, dinv, Wz, bz.reshape(1, OUT_CH),
      Wlz[:OUT_CH], blz.reshape(1, OUT_CH), Wh, bh.reshape(1, OUT_CH),
      Wlh[:OUT_CH], blh.reshape(1, OUT_CH))[0]

    sidx = jnp.concatenate(
        [samples, jnp.zeros((SP - S, 2), jnp.int32)]).T.reshape(2, ROWS_S, CH)
    g = _sample_sc(sidx, out_full)

    pred_full = pl.pallas_call(
        _head_body,
        grid=(SP // _BLK,),
        in_specs=[_rows((_BLK, OUT_CH)), _rows((_BLK, OUT_CH)),
                  _full((OUT_CH, HID)), _full((1, HID)),
                  _full((HID, 1)), _full((1, 1))],
        out_specs=[_rows((_BLK, 1))],
        out_shape=[jax.ShapeDtypeStruct((SP, 1), f32)],
    )(g[0], g[1], Wc1, bc1.reshape(1, HID), Wc2, bc2.reshape(1, 1))[0]

    return (pred_full[:S], out_full[:N])
